# Initial kernel scaffold; baseline (speedup 1.0000x reference)
import jax
import jax.numpy as jnp
from jax.experimental import pallas as pl


def kernel(input_ids, word_table, pos_table, tok_table, gamma, beta):
    raise NotImplementedError("write your pallas kernel here")


# speedup vs baseline: 2.5013x; 2.5013x over previous
import functools

import jax
import jax.numpy as jnp
from jax import lax
from jax.experimental import pallas as pl
from jax.experimental.pallas import tpu as pltpu
from jax.experimental.pallas import tpu_sc as plsc

VOCAB = 100000
DIM = 128
MAX_POS = 512
EPS = 1e-12
B, L = 1024, 200
N = B * L

NUM_CORES = 2
NUM_SUBCORES = 16
NW = NUM_CORES * NUM_SUBCORES
PER_W = N // NW
CHUNK = 128
NCHUNK = PER_W // CHUNK


def _allsum_vec(x):
    lanes = lax.iota(jnp.int32, 16)
    for s in (8, 4, 2, 1):
        x = x + x.at[lanes ^ s].get(mode="promise_in_bounds")
    return x


def _rsqrt_vec(v):
    i = lax.bitcast_convert_type(v, jnp.int32)
    i = jnp.int32(0x5F3759DF) - lax.shift_right_logical(i, 1)
    y = lax.bitcast_convert_type(i, jnp.float32)
    for _ in range(3):
        y = y * (1.5 - 0.5 * v * y * y)
    return y


@functools.partial(
    pl.kernel,
    out_type=jax.ShapeDtypeStruct((N, DIM), jnp.float32),
    mesh=plsc.VectorSubcoreMesh(core_axis_name="c", subcore_axis_name="s"),
    scratch_types=[
        pltpu.VMEM((CHUNK,), jnp.int32),
        pltpu.VMEM((CHUNK, DIM), jnp.float32),
        pltpu.VMEM((L, DIM), jnp.float32),
        pltpu.VMEM((DIM,), jnp.float32),
        pltpu.VMEM((DIM,), jnp.float32),
        pltpu.SemaphoreType.DMA,
    ],
)
def _embed_ln_sc(ids_hbm, word_hbm, bias_hbm, gamma_hbm, beta_hbm, out_hbm,
                 idx_v, rows_v, bias_v, gamma_v, beta_v, sem):
    wid = lax.axis_index("s") * NUM_CORES + lax.axis_index("c")
    base_w = wid * PER_W

    pltpu.sync_copy(bias_hbm, bias_v)
    pltpu.sync_copy(gamma_hbm, gamma_v)
    pltpu.sync_copy(beta_hbm, beta_v)

    g_vecs = [gamma_v[pl.ds(j * 16, 16)] for j in range(8)]
    b_vecs = [beta_v[pl.ds(j * 16, 16)] for j in range(8)]

    def chunk_body(g, carry):
        base = base_w + g * CHUNK
        pltpu.sync_copy(ids_hbm.at[pl.ds(base, CHUNK)], idx_v)
        pltpu.async_copy(word_hbm.at[idx_v], rows_v, sem).wait()

        def row_body(r, carry2):
            pos = lax.rem(base + r, L)
            acc = jnp.zeros((16,), jnp.float32)
            acc2 = jnp.zeros((16,), jnp.float32)
            for j in range(8):
                x = rows_v[r, pl.ds(j * 16, 16)] + bias_v[pos, pl.ds(j * 16, 16)]
                rows_v[r, pl.ds(j * 16, 16)] = x
                acc = acc + x
                acc2 = acc2 + x * x
            s1 = _allsum_vec(acc)
            s2 = _allsum_vec(acc2)
            mean_v = s1 * (1.0 / DIM)
            var_v = s2 * (1.0 / DIM) - mean_v * mean_v + EPS
            inv_v = _rsqrt_vec(var_v)
            d_v = mean_v * inv_v
            for j in range(8):
                x = rows_v[r, pl.ds(j * 16, 16)]
                rows_v[r, pl.ds(j * 16, 16)] = (x * inv_v - d_v) * g_vecs[j] + b_vecs[j]
            return carry2

        lax.fori_loop(0, CHUNK, row_body, 0)
        pltpu.sync_copy(rows_v, out_hbm.at[pl.ds(base, CHUNK)])
        return carry

    lax.fori_loop(0, NCHUNK, chunk_body, 0)


def kernel(input_ids, word_table, pos_table, tok_table, gamma, beta):
    ids_flat = input_ids.reshape(-1)
    bias = pos_table[:L] + tok_table[0]
    out = _embed_ln_sc(ids_flat, word_table, bias, gamma, beta)
    return out.reshape(B, L, DIM)


# speedup vs baseline: 2.9152x; 1.1655x over previous
import functools

import jax
import jax.numpy as jnp
from jax import lax
from jax.experimental import pallas as pl
from jax.experimental.pallas import tpu as pltpu
from jax.experimental.pallas import tpu_sc as plsc

VOCAB = 100000
DIM = 128
MAX_POS = 512
EPS = 1e-12
B, L = 1024, 200
N = B * L

NUM_CORES = 2
NUM_SUBCORES = 16
NW = NUM_CORES * NUM_SUBCORES
PER_W = N // NW
CHUNK = 128
NCHUNK = PER_W // CHUNK


def _allsum_vec(x):
    lanes = lax.iota(jnp.int32, 16)
    for s in (8, 4, 2, 1):
        x = x + x.at[lanes ^ s].get(mode="promise_in_bounds")
    return x


def _rsqrt_vec(v):
    i = lax.bitcast_convert_type(v, jnp.int32)
    i = jnp.int32(0x5F3759DF) - lax.shift_right_logical(i, 1)
    y = lax.bitcast_convert_type(i, jnp.float32)
    for _ in range(3):
        y = y * (1.5 - 0.5 * v * y * y)
    return y


@functools.partial(
    pl.kernel,
    out_type=jax.ShapeDtypeStruct((N, DIM), jnp.float32),
    mesh=plsc.VectorSubcoreMesh(core_axis_name="c", subcore_axis_name="s"),
    scratch_types=[
        pltpu.VMEM((NCHUNK, CHUNK), jnp.int32),
        pltpu.VMEM((CHUNK, DIM), jnp.float32),
        pltpu.VMEM((CHUNK, DIM), jnp.float32),
        pltpu.VMEM((L, DIM), jnp.float32),
        pltpu.VMEM((DIM,), jnp.float32),
        pltpu.VMEM((DIM,), jnp.float32),
        pltpu.SemaphoreType.DMA,
        pltpu.SemaphoreType.DMA,
        pltpu.SemaphoreType.DMA,
        pltpu.SemaphoreType.DMA,
    ],
)
def _embed_ln_sc(ids_hbm, word_hbm, bias_hbm, gamma_hbm, beta_hbm, out_hbm,
                 idx_all, rows0, rows1, bias_v, gamma_v, beta_v,
                 gsem0, gsem1, osem0, osem1):
    wid = lax.axis_index("s") * NUM_CORES + lax.axis_index("c")
    base_w = wid * PER_W
    rows_bufs = (rows0, rows1)
    gsems = (gsem0, gsem1)
    osems = (osem0, osem1)

    pltpu.sync_copy(ids_hbm.at[wid], idx_all)
    pltpu.sync_copy(bias_hbm, bias_v)
    pltpu.sync_copy(gamma_hbm, gamma_v)
    pltpu.sync_copy(beta_hbm, beta_v)

    g_vecs = [gamma_v[pl.ds(j * 16, 16)] for j in range(8)]
    b_vecs = [beta_v[pl.ds(j * 16, 16)] for j in range(8)]

    def row_ln(rows_v, base, r):
        pos = lax.rem(base + r, L)
        acc = jnp.zeros((16,), jnp.float32)
        acc2 = jnp.zeros((16,), jnp.float32)
        for j in range(8):
            x = rows_v[r, pl.ds(j * 16, 16)] + bias_v[pos, pl.ds(j * 16, 16)]
            rows_v[r, pl.ds(j * 16, 16)] = x
            acc = acc + x
            acc2 = acc2 + x * x
        s1 = _allsum_vec(acc)
        s2 = _allsum_vec(acc2)
        mean_v = s1 * (1.0 / DIM)
        var_v = s2 * (1.0 / DIM) - mean_v * mean_v + EPS
        inv_v = _rsqrt_vec(var_v)
        d_v = mean_v * inv_v
        for j in range(8):
            x = rows_v[r, pl.ds(j * 16, 16)]
            rows_v[r, pl.ds(j * 16, 16)] = (x * inv_v - d_v) * g_vecs[j] + b_vecs[j]

    def start_gather(g, b):
        pltpu.async_copy(word_hbm.at[idx_all.at[g]], rows_bufs[b], gsems[b])

    def wait_gather(g, b):
        pltpu.make_async_copy(word_hbm.at[idx_all.at[g]], rows_bufs[b],
                              gsems[b]).wait()

    def out_desc(base, b):
        return pltpu.make_async_copy(rows_bufs[b],
                                     out_hbm.at[pl.ds(base, CHUNK)], osems[b])

    start_gather(0, 0)

    def outer_body(i, carry):
        g0 = i * 2
        for db in range(2):
            g = g0 + db
            base = base_w + g * CHUNK
            gn = g + 1
            nb = 1 - db

            @pl.when(gn < NCHUNK)
            def _():
                @pl.when(g >= 1)
                def _():
                    out_desc(base_w, nb).wait()
                start_gather(gn, nb)

            wait_gather(g, db)
            def row2(k, c2):
                row_ln(rows_bufs[db], base, 2 * k)
                row_ln(rows_bufs[db], base, 2 * k + 1)
                return c2
            lax.fori_loop(0, CHUNK // 2, row2, 0)
            pltpu.async_copy(rows_bufs[db], out_hbm.at[pl.ds(base, CHUNK)],
                             osems[db])
        return carry

    lax.fori_loop(0, NCHUNK // 2, outer_body, 0)
    out_desc(base_w, 0).wait()
    out_desc(base_w, 1).wait()


def kernel(input_ids, word_table, pos_table, tok_table, gamma, beta):
    ids_2d = input_ids.reshape(NW, NCHUNK, CHUNK)
    bias = pos_table[:L] + tok_table[0]
    out = _embed_ln_sc(ids_2d, word_table, bias, gamma, beta)
    return out.reshape(B, L, DIM)


# speedup vs baseline: 4.2789x; 1.4678x over previous
import functools

import jax
import jax.numpy as jnp
from jax import lax
from jax.experimental import pallas as pl
from jax.experimental.pallas import tpu as pltpu
from jax.experimental.pallas import tpu_sc as plsc

VOCAB = 100000
DIM = 128
MAX_POS = 512
EPS = 1e-12
B, L = 1024, 200
N = B * L

NUM_CORES = 2
NUM_SUBCORES = 16
NW = NUM_CORES * NUM_SUBCORES
PER_W = N // NW
CHUNK = 128
NCHUNK = PER_W // CHUNK


def _allsum_vec(x):
    lanes = lax.iota(jnp.int32, 16)
    for s in (8, 4, 2, 1):
        x = x + x.at[lanes ^ s].get(mode="promise_in_bounds")
    return x


def _rsqrt_vec(v):
    i = lax.bitcast_convert_type(v, jnp.int32)
    i = jnp.int32(0x5F3759DF) - lax.shift_right_logical(i, 1)
    y = lax.bitcast_convert_type(i, jnp.float32)
    for _ in range(2):
        y = y * (1.5 - 0.5 * v * y * y)
    return y


@functools.partial(
    pl.kernel,
    out_type=jax.ShapeDtypeStruct((N, DIM), jnp.float32),
    mesh=plsc.VectorSubcoreMesh(core_axis_name="c", subcore_axis_name="s"),
    scratch_types=[
        pltpu.VMEM((NCHUNK, CHUNK), jnp.int32),
        pltpu.VMEM((CHUNK, DIM), jnp.float32),
        pltpu.VMEM((CHUNK, DIM), jnp.float32),
        pltpu.VMEM((L, DIM), jnp.float32),
        pltpu.VMEM((DIM,), jnp.float32),
        pltpu.VMEM((DIM,), jnp.float32),
        pltpu.SemaphoreType.DMA,
        pltpu.SemaphoreType.DMA,
        pltpu.SemaphoreType.DMA,
        pltpu.SemaphoreType.DMA,
    ],
)
def _embed_ln_sc(ids_hbm, word_hbm, bias_hbm, gamma_hbm, beta_hbm, out_hbm,
                 idx_all, rows0, rows1, bias_v, gamma_v, beta_v,
                 gsem0, gsem1, osem0, osem1):
    wid = lax.axis_index("s") * NUM_CORES + lax.axis_index("c")
    base_w = wid * PER_W
    rows_bufs = (rows0, rows1)
    gsems = (gsem0, gsem1)
    osems = (osem0, osem1)

    pltpu.sync_copy(ids_hbm.at[wid], idx_all)
    pltpu.sync_copy(bias_hbm, bias_v)
    pltpu.sync_copy(gamma_hbm, gamma_v)
    pltpu.sync_copy(beta_hbm, beta_v)

    g_vecs = [gamma_v[pl.ds(j * 16, 16)] for j in range(8)]
    b_vecs = [beta_v[pl.ds(j * 16, 16)] for j in range(8)]

    def row_ln(rows_v, base, r):
        pos = lax.rem(base + r, L)
        xs = []
        acc = jnp.zeros((16,), jnp.float32)
        acc2 = jnp.zeros((16,), jnp.float32)
        for j in range(8):
            x = rows_v[r, pl.ds(j * 16, 16)] + bias_v[pos, pl.ds(j * 16, 16)]
            xs.append(x)
            acc = acc + x
            acc2 = acc2 + x * x
        s1 = _allsum_vec(acc)
        s2 = _allsum_vec(acc2)
        mean_v = s1 * (1.0 / DIM)
        var_v = s2 * (1.0 / DIM) - mean_v * mean_v + EPS
        inv_v = _rsqrt_vec(var_v)
        d_v = mean_v * inv_v
        for j in range(8):
            rows_v[r, pl.ds(j * 16, 16)] = (xs[j] * inv_v - d_v) * g_vecs[j] + b_vecs[j]

    def start_gather(g, b):
        pltpu.async_copy(word_hbm.at[idx_all.at[g]], rows_bufs[b], gsems[b])

    def wait_gather(g, b):
        pltpu.make_async_copy(word_hbm.at[idx_all.at[g]], rows_bufs[b],
                              gsems[b]).wait()

    def out_desc(base, b):
        return pltpu.make_async_copy(rows_bufs[b],
                                     out_hbm.at[pl.ds(base, CHUNK)], osems[b])

    start_gather(0, 0)

    def outer_body(i, carry):
        g0 = i * 2
        for db in range(2):
            g = g0 + db
            base = base_w + g * CHUNK
            gn = g + 1
            nb = 1 - db

            @pl.when(gn < NCHUNK)
            def _():
                @pl.when(g >= 1)
                def _():
                    out_desc(base_w, nb).wait()
                start_gather(gn, nb)

            wait_gather(g, db)
            def row2(k, c2):
                row_ln(rows_bufs[db], base, 2 * k)
                row_ln(rows_bufs[db], base, 2 * k + 1)
                return c2
            lax.fori_loop(0, CHUNK // 2, row2, 0)
            pltpu.async_copy(rows_bufs[db], out_hbm.at[pl.ds(base, CHUNK)],
                             osems[db])
        return carry

    lax.fori_loop(0, NCHUNK // 2, outer_body, 0)
    out_desc(base_w, 0).wait()
    out_desc(base_w, 1).wait()


def kernel(input_ids, word_table, pos_table, tok_table, gamma, beta):
    ids_2d = input_ids.reshape(NW, NCHUNK, CHUNK)
    bias = pos_table[:L] + tok_table[0]
    out = _embed_ln_sc(ids_2d, word_table, bias, gamma, beta)
    return out.reshape(B, L, DIM)


# speedup vs baseline: 4.3471x; 1.0159x over previous
import functools

import jax
import jax.numpy as jnp
from jax import lax
from jax.experimental import pallas as pl
from jax.experimental.pallas import tpu as pltpu
from jax.experimental.pallas import tpu_sc as plsc

VOCAB = 100000
DIM = 128
MAX_POS = 512
EPS = 1e-12
B, L = 1024, 200
N = B * L

NUM_CORES = 2
NUM_SUBCORES = 16
NW = NUM_CORES * NUM_SUBCORES
PER_W = N // NW
CHUNK = 128
NCHUNK = PER_W // CHUNK


def _allsum_vec(x):
    lanes = lax.iota(jnp.int32, 16)
    for s in (8, 4, 2, 1):
        x = x + x.at[lanes ^ s].get(mode="promise_in_bounds")
    return x


def _rsqrt_vec(v):
    i = lax.bitcast_convert_type(v, jnp.int32)
    i = jnp.int32(0x5F3759DF) - lax.shift_right_logical(i, 1)
    y = lax.bitcast_convert_type(i, jnp.float32)
    for _ in range(2):
        y = y * (1.5 - 0.5 * v * y * y)
    return y


@functools.partial(
    pl.kernel,
    out_type=jax.ShapeDtypeStruct((N, DIM), jnp.float32),
    mesh=plsc.VectorSubcoreMesh(core_axis_name="c", subcore_axis_name="s"),
    scratch_types=[
        pltpu.VMEM((NCHUNK, CHUNK), jnp.int32),
        pltpu.VMEM((CHUNK, DIM), jnp.float32),
        pltpu.VMEM((CHUNK, DIM), jnp.float32),
        pltpu.VMEM((L, DIM), jnp.float32),
        pltpu.VMEM((DIM,), jnp.float32),
        pltpu.VMEM((DIM,), jnp.float32),
        pltpu.SemaphoreType.DMA,
        pltpu.SemaphoreType.DMA,
        pltpu.SemaphoreType.DMA,
        pltpu.SemaphoreType.DMA,
    ],
)
def _embed_ln_sc(ids_hbm, word_hbm, bias_hbm, gamma_hbm, beta_hbm, out_hbm,
                 idx_all, rows0, rows1, bias_v, gamma_v, beta_v,
                 gsem0, gsem1, osem0, osem1):
    wid = lax.axis_index("s") * NUM_CORES + lax.axis_index("c")
    base_w = wid * PER_W
    rows_bufs = (rows0, rows1)
    gsems = (gsem0, gsem1)
    osems = (osem0, osem1)

    pltpu.sync_copy(ids_hbm.at[wid], idx_all)
    pltpu.sync_copy(bias_hbm, bias_v)
    pltpu.sync_copy(gamma_hbm, gamma_v)
    pltpu.sync_copy(beta_hbm, beta_v)

    g_vecs = [gamma_v[pl.ds(j * 16, 16)] for j in range(8)]
    b_vecs = [beta_v[pl.ds(j * 16, 16)] for j in range(8)]

    def row_ln(rows_v, base, r):
        pos = lax.rem(base + r, L)
        xs = []
        acc = jnp.zeros((16,), jnp.float32)
        acc2 = jnp.zeros((16,), jnp.float32)
        for j in range(8):
            x = rows_v[r, pl.ds(j * 16, 16)] + bias_v[pos, pl.ds(j * 16, 16)]
            xs.append(x)
            acc = acc + x
            acc2 = acc2 + x * x
        s1 = _allsum_vec(acc)
        s2 = _allsum_vec(acc2)
        mean_v = s1 * (1.0 / DIM)
        var_v = s2 * (1.0 / DIM) - mean_v * mean_v + EPS
        inv_v = _rsqrt_vec(var_v)
        d_v = mean_v * inv_v
        for j in range(8):
            rows_v[r, pl.ds(j * 16, 16)] = (xs[j] * inv_v - d_v) * g_vecs[j] + b_vecs[j]

    def start_gather(g, b):
        pltpu.async_copy(word_hbm.at[idx_all.at[g]], rows_bufs[b], gsems[b])

    def wait_gather(g, b):
        pltpu.make_async_copy(word_hbm.at[idx_all.at[g]], rows_bufs[b],
                              gsems[b]).wait()

    def out_desc(base, b):
        return pltpu.make_async_copy(rows_bufs[b],
                                     out_hbm.at[pl.ds(base, CHUNK)], osems[b])

    start_gather(0, 0)

    def outer_body(i, carry):
        g0 = i * 2
        for db in range(2):
            g = g0 + db
            base = base_w + g * CHUNK
            gn = g + 1
            nb = 1 - db

            @pl.when(gn < NCHUNK)
            def _():
                @pl.when(g >= 1)
                def _():
                    out_desc(base_w, nb).wait()
                start_gather(gn, nb)

            wait_gather(g, db)
            def row4(k, c2):
                for u in range(4):
                    row_ln(rows_bufs[db], base, 4 * k + u)
                return c2
            lax.fori_loop(0, CHUNK // 4, row4, 0)
            pltpu.async_copy(rows_bufs[db], out_hbm.at[pl.ds(base, CHUNK)],
                             osems[db])
        return carry

    lax.fori_loop(0, NCHUNK // 2, outer_body, 0)
    out_desc(base_w, 0).wait()
    out_desc(base_w, 1).wait()


def kernel(input_ids, word_table, pos_table, tok_table, gamma, beta):
    ids_2d = input_ids.reshape(NW, NCHUNK, CHUNK)
    bias = pos_table[:L] + tok_table[0]
    out = _embed_ln_sc(ids_2d, word_table, bias, gamma, beta)
    return out.reshape(B, L, DIM)


# speedup vs baseline: 7.9526x; 1.8294x over previous
import functools

import jax
import jax.numpy as jnp
from jax import lax
from jax.experimental import pallas as pl
from jax.experimental.pallas import tpu as pltpu
from jax.experimental.pallas import tpu_sc as plsc

VOCAB = 100000
DIM = 128
MAX_POS = 512
EPS = 1e-12
B, L = 1024, 200
N = B * L

NUM_CORES = 2
NUM_SUBCORES = 16
NW = NUM_CORES * NUM_SUBCORES
PER_W = N // NW
CHUNK = 128
NCHUNK = PER_W // CHUNK


def _allsum_vec(x):
    lanes = lax.iota(jnp.int32, 16)
    for s in (8, 4, 2, 1):
        x = x + x.at[lanes ^ s].get(mode="promise_in_bounds")
    return x


def _rsqrt_vec(v):
    i = lax.bitcast_convert_type(v, jnp.int32)
    i = jnp.int32(0x5F3759DF) - lax.shift_right_logical(i, 1)
    y = lax.bitcast_convert_type(i, jnp.float32)
    for _ in range(2):
        y = y * (1.5 - 0.5 * v * y * y)
    return y


@functools.partial(
    pl.kernel,
    out_type=jax.ShapeDtypeStruct((N, DIM), jnp.float32),
    mesh=plsc.VectorSubcoreMesh(core_axis_name="c", subcore_axis_name="s"),
    scratch_types=[
        pltpu.VMEM((NCHUNK, CHUNK), jnp.int32),
        pltpu.VMEM((CHUNK, DIM), jnp.float32),
        pltpu.VMEM((CHUNK, DIM), jnp.float32),
        pltpu.VMEM((L, DIM), jnp.float32),
        pltpu.VMEM((DIM,), jnp.float32),
        pltpu.VMEM((DIM,), jnp.float32),
        pltpu.SemaphoreType.DMA,
        pltpu.SemaphoreType.DMA,
        pltpu.SemaphoreType.DMA,
        pltpu.SemaphoreType.DMA,
    ],
)
def _embed_ln_sc(ids_hbm, word_hbm, bias_hbm, gamma_hbm, beta_hbm, out_hbm,
                 idx_all, rows0, rows1, bias_v, gamma_v, beta_v,
                 gsem0, gsem1, osem0, osem1):
    wid = lax.axis_index("s") * NUM_CORES + lax.axis_index("c")
    base_w = wid * PER_W
    rows_bufs = (rows0, rows1)
    gsems = (gsem0, gsem1)
    osems = (osem0, osem1)

    pltpu.sync_copy(ids_hbm.at[wid], idx_all)
    pltpu.sync_copy(bias_hbm, bias_v)
    pltpu.sync_copy(gamma_hbm, gamma_v)
    pltpu.sync_copy(beta_hbm, beta_v)

    g_vecs = [gamma_v[pl.ds(j * 16, 16)] for j in range(8)]
    b_vecs = [beta_v[pl.ds(j * 16, 16)] for j in range(8)]

    def row_ln(rows_v, base, r):
        pos = lax.rem(base + r, L)
        xs = []
        acc = jnp.zeros((16,), jnp.float32)
        acc2 = jnp.zeros((16,), jnp.float32)
        for j in range(8):
            x = rows_v[r, pl.ds(j * 16, 16)] + bias_v[pos, pl.ds(j * 16, 16)]
            xs.append(x)
            acc = acc + x
            acc2 = acc2 + x * x
        s1 = _allsum_vec(acc)
        s2 = _allsum_vec(acc2)
        mean_v = s1 * (1.0 / DIM)
        var_v = s2 * (1.0 / DIM) - mean_v * mean_v + EPS
        inv_v = _rsqrt_vec(var_v)
        d_v = mean_v * inv_v
        for j in range(8):
            rows_v[r, pl.ds(j * 16, 16)] = (xs[j] * inv_v - d_v) * g_vecs[j] + b_vecs[j]

    def start_gather(g, b):
        pltpu.async_copy(word_hbm.at[idx_all.at[g]], rows_bufs[b], gsems[b])

    def wait_gather(g, b):
        pltpu.make_async_copy(word_hbm.at[idx_all.at[g]], rows_bufs[b],
                              gsems[b]).wait()

    def out_desc(base, b):
        return pltpu.make_async_copy(rows_bufs[b],
                                     out_hbm.at[pl.ds(base, CHUNK)], osems[b])

    start_gather(0, 0)

    def outer_body(i, carry):
        g0 = i * 2
        for db in range(2):
            g = g0 + db
            base = base_w + g * CHUNK
            gn = g + 1
            nb = 1 - db

            @pl.when(gn < NCHUNK)
            def _():
                @pl.when(g >= 1)
                def _():
                    out_desc(base_w, nb).wait()
                start_gather(gn, nb)

            wait_gather(g, db)
            @plsc.parallel_loop(0, CHUNK, 1, unroll=4)
            def _(r):
                row_ln(rows_bufs[db], base, r)
            pltpu.async_copy(rows_bufs[db], out_hbm.at[pl.ds(base, CHUNK)],
                             osems[db])
        return carry

    lax.fori_loop(0, NCHUNK // 2, outer_body, 0)
    out_desc(base_w, 0).wait()
    out_desc(base_w, 1).wait()


def kernel(input_ids, word_table, pos_table, tok_table, gamma, beta):
    ids_2d = input_ids.reshape(NW, NCHUNK, CHUNK)
    bias = pos_table[:L] + tok_table[0]
    out = _embed_ln_sc(ids_2d, word_table, bias, gamma, beta)
    return out.reshape(B, L, DIM)
